# baseline (speedup 1.0000x reference)
import functools

import jax
import jax.numpy as jnp
from jax import lax
from jax.experimental import pallas as pl
from jax.experimental.pallas import tpu as pltpu
from jax.experimental.pallas import tpu_sc as plsc

N = 10000
D = 128
E = 640000
P = 100000
EPS = 1e-5

NC = 2
NS = 16
NW = NC * NS

SEG_C = 128
SEG_CHUNKS = 160
EPT = SEG_C * SEG_CHUNKS
EPAD = NW * EPT
NPAD = 10240
RPT = NPAD // NS
ZITER = RPT // SEG_C

PPAD = 102400
PPT = PPAD // NW
PG_C = 128
PG_CHUNKS = PPT // PG_C

_SC_PARAMS = pltpu.CompilerParams(use_tc_tiling_on_sc=False,
                                  needs_layout_passes=False)


def _sc_segsum_body(compute_deg, *refs):
    if compute_deg:
        (feats, src, dst, zfeat, parts, degflat,
         acc, idx_s, idx_d, rows, dpriv, sem) = refs
    else:
        (feats, src, dst, zfeat, parts,
         acc, idx_s, idx_d, rows, sem) = refs
    cid = lax.axis_index("c")
    sid = lax.axis_index("s")
    wid = cid * NS + sid
    base = wid * EPT

    pltpu.sync_copy(zfeat, rows)
    for j in range(ZITER):
        pltpu.sync_copy(rows, acc.at[pl.ds(sid * RPT + j * SEG_C, SEG_C)])
    if compute_deg:
        def zero_deg(i, carry):
            dpriv[pl.ds(i * 16, 16)] = jnp.zeros((16,), jnp.float32)
            return carry
        lax.fori_loop(0, NPAD // 16, zero_deg, 0)
    plsc.subcore_barrier()

    ones16 = jnp.ones((16,), jnp.float32)

    def chunk(g, carry):
        off = pl.multiple_of(base + g * SEG_C, 8)
        pltpu.sync_copy(src.at[pl.ds(off, SEG_C)], idx_s)
        pltpu.sync_copy(dst.at[pl.ds(off, SEG_C)], idx_d)
        pltpu.async_copy(feats.at[idx_s], rows, sem).wait()
        pltpu.sync_copy(rows, acc.at[idx_d], add=True)
        if compute_deg:
            for k in range(SEG_C // 16):
                dv = idx_d[pl.ds(k * 16, 16)]
                plsc.addupdate_scatter(dpriv, [dv], ones16)
        return carry

    lax.fori_loop(0, SEG_CHUNKS, chunk, 0)
    plsc.subcore_barrier()

    for j in range(ZITER):
        sl = pl.ds(sid * RPT + j * SEG_C, SEG_C)
        pltpu.sync_copy(acc.at[sl], rows)
        pltpu.sync_copy(rows, parts.at[cid].at[sl])
    if compute_deg:
        pltpu.sync_copy(dpriv, degflat.at[pl.ds(wid * NPAD, NPAD)])


@functools.lru_cache(maxsize=None)
def _get_segsum(compute_deg):
    mesh = plsc.VectorSubcoreMesh(core_axis_name="c", subcore_axis_name="s")
    out_type = [jax.ShapeDtypeStruct((NC, NPAD, D), jnp.float32)]
    if compute_deg:
        out_type.append(jax.ShapeDtypeStruct((NW * NPAD,), jnp.float32))
    scratch = [
        pltpu.VMEM_SHARED((NPAD, D), jnp.float32),
        pltpu.VMEM((SEG_C,), jnp.int32),
        pltpu.VMEM((SEG_C,), jnp.int32),
        pltpu.VMEM((SEG_C, D), jnp.float32),
    ]
    if compute_deg:
        scratch.append(pltpu.VMEM((NPAD,), jnp.float32))
    scratch.append(pltpu.SemaphoreType.DMA)
    return pl.kernel(
        functools.partial(_sc_segsum_body, compute_deg),
        out_type=out_type,
        mesh=mesh,
        scratch_types=scratch,
        compiler_params=_SC_PARAMS,
    )


def _sc_pairs_body(u, v, p0, p1, gu, gv, idx0, idx1, bufu, bufv, sem):
    wid = lax.axis_index("c") * NS + lax.axis_index("s")
    base = wid * PPT

    def chunk(g, carry):
        off = pl.multiple_of(base + g * PG_C, 8)
        sl = pl.ds(off, PG_C)
        pltpu.sync_copy(p0.at[sl], idx0)
        pltpu.sync_copy(p1.at[sl], idx1)
        pltpu.async_copy(u.at[idx0], bufu, sem).wait()
        pltpu.sync_copy(bufu, gu.at[sl])
        pltpu.async_copy(v.at[idx1], bufv, sem).wait()
        pltpu.sync_copy(bufv, gv.at[sl])
        return carry

    lax.fori_loop(0, PG_CHUNKS, chunk, 0)


@functools.lru_cache(maxsize=None)
def _get_pair_gather():
    mesh = plsc.VectorSubcoreMesh(core_axis_name="c", subcore_axis_name="s")
    return pl.kernel(
        _sc_pairs_body,
        out_type=[
            jax.ShapeDtypeStruct((PPAD, D), jnp.float32),
            jax.ShapeDtypeStruct((PPAD, D), jnp.float32),
        ],
        mesh=mesh,
        scratch_types=[
            pltpu.VMEM((PG_C,), jnp.int32),
            pltpu.VMEM((PG_C,), jnp.int32),
            pltpu.VMEM((PG_C, D), jnp.float32),
            pltpu.VMEM((PG_C, D), jnp.float32),
            pltpu.SemaphoreType.DMA,
        ],
        compiler_params=_SC_PARAMS,
    )


def _dot_t(a, b):
    return lax.dot_general(a, b, (((1,), (1,)), ((), ())),
                           preferred_element_type=jnp.float32)


def _tc_layer1_body(parts, dcol, x, w1l, b1l, w1r, gamma, beta, h):
    s = parts[0, :N] + parts[1, :N]
    agg = s / jnp.maximum(dcol[...], 1.0)
    t = _dot_t(agg, w1l[...]) + b1l[...] + _dot_t(x[...], w1r[...])
    mu = jnp.mean(t, axis=0, keepdims=True)
    var = jnp.mean((t - mu) ** 2, axis=0, keepdims=True)
    hh = gamma[...] * (t - mu) * lax.rsqrt(var + EPS) + beta[...]
    h[...] = jnp.maximum(hh, 0.0)


def _tc_layer2_body(parts, dcol, h, w2l, b2l, w2r, w3a, w3b, b3, u, v):
    s = parts[0, :N] + parts[1, :N]
    agg = s / jnp.maximum(dcol[...], 1.0)
    z = _dot_t(agg, w2l[...]) + b2l[...] + _dot_t(h[...], w2r[...])
    u[...] = _dot_t(z, w3a[...]) + b3[...]
    v[...] = _dot_t(z, w3b[...])


HBLK = 2048


def _tc_head_body(gu, gv, w4p, b4p, w5a, o):
    h1 = jnp.maximum(gu[...] + gv[...], 0.0)
    h2 = jnp.maximum(_dot_t(h1, w4p[...]) + b4p[...], 0.0)
    o[...] = jax.nn.sigmoid(_dot_t(h2, w5a[...]))


def kernel(x, edge_index, edge_pairs, W1l, b1l, W1r, gamma, beta,
           W2l, b2l, W2r, W3, b3, W4, b4, W5, b5):
    npd = EPAD - E
    src = jnp.concatenate([edge_index[0], jnp.zeros((npd,), jnp.int32)])
    dst = jnp.concatenate(
        [edge_index[1], N + jnp.arange(npd, dtype=jnp.int32) % (NPAD - N)])
    zfeat = jnp.zeros((SEG_C, D), jnp.float32)

    parts1, degflat = _get_segsum(True)(x, src, dst, zfeat)
    dcol = degflat.reshape(NW, NPAD).sum(axis=0)[:N, None]

    h = pl.pallas_call(
        _tc_layer1_body,
        out_shape=jax.ShapeDtypeStruct((N, D), jnp.float32),
    )(parts1, dcol, x, W1l, b1l.reshape(1, D), W1r,
      gamma.reshape(1, D), beta.reshape(1, D))

    (parts2,) = _get_segsum(False)(h, src, dst, zfeat)

    u, v = pl.pallas_call(
        _tc_layer2_body,
        out_shape=[
            jax.ShapeDtypeStruct((N, D), jnp.float32),
            jax.ShapeDtypeStruct((N, D), jnp.float32),
        ],
    )(parts2, dcol, h, W2l, b2l.reshape(1, D), W2r,
      W3[:, :D], W3[:, D:], b3.reshape(1, D))

    pad = jnp.zeros((PPAD - P,), jnp.int32)
    p0 = jnp.concatenate([edge_pairs[0], pad])
    p1 = jnp.concatenate([edge_pairs[1], pad])
    gu, gv = _get_pair_gather()(u, v, p0, p1)

    w4p = jnp.concatenate([W4, jnp.zeros((D // 2, D), jnp.float32)], axis=0)
    b4p = jnp.concatenate(
        [b4, jnp.ones((1,), jnp.float32),
         jnp.zeros((D // 2 - 1,), jnp.float32)]).reshape(1, D)
    w5a = jnp.concatenate(
        [W5, b5.reshape(1, 1), jnp.zeros((1, D // 2 - 1), jnp.float32)],
        axis=1)

    o = pl.pallas_call(
        _tc_head_body,
        grid=(PPAD // HBLK,),
        in_specs=[
            pl.BlockSpec((HBLK, D), lambda i: (i, 0)),
            pl.BlockSpec((HBLK, D), lambda i: (i, 0)),
            pl.BlockSpec((D, D), lambda i: (0, 0)),
            pl.BlockSpec((1, D), lambda i: (0, 0)),
            pl.BlockSpec((1, D), lambda i: (0, 0)),
        ],
        out_specs=pl.BlockSpec((HBLK, 1), lambda i: (i, 0)),
        out_shape=jax.ShapeDtypeStruct((PPAD, 1), jnp.float32),
    )(gu, gv, w4p, b4p, w5a)

    return o[:P]


# speedup vs baseline: 1.8333x; 1.8333x over previous
import functools

import jax
import jax.numpy as jnp
from jax import lax
from jax.experimental import pallas as pl
from jax.experimental.pallas import tpu as pltpu
from jax.experimental.pallas import tpu_sc as plsc

N = 10000
D = 128
E = 640000
P = 100000
EPS = 1e-5

NC = 2
NS = 16
NW = NC * NS

SEG_C = 128
SEG_CHUNKS = 160
EPT = SEG_C * SEG_CHUNKS
EPAD = NW * EPT
NPAD = 10240
RPT = NPAD // NS
ZITER = RPT // SEG_C

PPAD = 102400
PPT = PPAD // NW
PG_C = 128
PG_CHUNKS = PPT // PG_C

_SC_PARAMS = pltpu.CompilerParams(use_tc_tiling_on_sc=False,
                                  needs_layout_passes=False)


def _sc_segsum_body(compute_deg, *refs):
    if compute_deg:
        (feats, src, dst, zfeat, parts, degflat,
         acc, idx_s, idx_d, rows, dpriv, sem) = refs
    else:
        (feats, src, dst, zfeat, parts,
         acc, idx_s, idx_d, rows, sem) = refs
    cid = lax.axis_index("c")
    sid = lax.axis_index("s")
    wid = cid * NS + sid
    base = wid * EPT

    pltpu.sync_copy(zfeat, rows)
    for j in range(ZITER):
        pltpu.sync_copy(rows, acc.at[pl.ds(sid * RPT + j * SEG_C, SEG_C)])
    if compute_deg:
        def zero_deg(i, carry):
            dpriv[pl.ds(i * 16, 16)] = jnp.zeros((16,), jnp.float32)
            return carry
        lax.fori_loop(0, NPAD // 16, zero_deg, 0)
    plsc.subcore_barrier()

    ones16 = jnp.ones((16,), jnp.float32)

    def chunk(g, carry):
        off = pl.multiple_of(base + g * SEG_C, 8)
        pltpu.sync_copy(src.at[pl.ds(off, SEG_C)], idx_s)
        pltpu.sync_copy(dst.at[pl.ds(off, SEG_C)], idx_d)
        pltpu.async_copy(feats.at[idx_s], rows, sem).wait()
        pltpu.sync_copy(rows, acc.at[idx_d], add=True)
        if compute_deg:
            for k in range(SEG_C // 16):
                dv = idx_d[pl.ds(k * 16, 16)]
                plsc.addupdate_scatter(dpriv, [dv], ones16)
        return carry

    lax.fori_loop(0, SEG_CHUNKS, chunk, 0)
    plsc.subcore_barrier()

    for j in range(ZITER):
        sl = pl.ds(sid * RPT + j * SEG_C, SEG_C)
        pltpu.sync_copy(acc.at[sl], rows)
        pltpu.sync_copy(rows, parts.at[cid].at[sl])
    if compute_deg:
        pltpu.sync_copy(dpriv, degflat.at[pl.ds(wid * NPAD, NPAD)])


@functools.lru_cache(maxsize=None)
def _get_segsum(compute_deg):
    mesh = plsc.VectorSubcoreMesh(core_axis_name="c", subcore_axis_name="s")
    out_type = [jax.ShapeDtypeStruct((NC, NPAD, D), jnp.float32)]
    if compute_deg:
        out_type.append(jax.ShapeDtypeStruct((NW * NPAD,), jnp.float32))
    scratch = [
        pltpu.VMEM_SHARED((NPAD, D), jnp.float32),
        pltpu.VMEM((SEG_C,), jnp.int32),
        pltpu.VMEM((SEG_C,), jnp.int32),
        pltpu.VMEM((SEG_C, D), jnp.float32),
    ]
    if compute_deg:
        scratch.append(pltpu.VMEM((NPAD,), jnp.float32))
    scratch.append(pltpu.SemaphoreType.DMA)
    return pl.kernel(
        functools.partial(_sc_segsum_body, compute_deg),
        out_type=out_type,
        mesh=mesh,
        scratch_types=scratch,
        compiler_params=_SC_PARAMS,
    )


def _sc_pairs_body(u, v, p0, p1, gu, gv, idx0, idx1, bufu, bufv, sem):
    wid = lax.axis_index("c") * NS + lax.axis_index("s")
    base = wid * PPT

    def chunk(g, carry):
        off = pl.multiple_of(base + g * PG_C, 8)
        sl = pl.ds(off, PG_C)
        pltpu.sync_copy(p0.at[sl], idx0)
        pltpu.sync_copy(p1.at[sl], idx1)
        pltpu.async_copy(u.at[idx0], bufu, sem).wait()
        pltpu.sync_copy(bufu, gu.at[sl])
        pltpu.async_copy(v.at[idx1], bufv, sem).wait()
        pltpu.sync_copy(bufv, gv.at[sl])
        return carry

    lax.fori_loop(0, PG_CHUNKS, chunk, 0)


@functools.lru_cache(maxsize=None)
def _get_pair_gather():
    mesh = plsc.VectorSubcoreMesh(core_axis_name="c", subcore_axis_name="s")
    return pl.kernel(
        _sc_pairs_body,
        out_type=[
            jax.ShapeDtypeStruct((PPAD, D), jnp.float32),
            jax.ShapeDtypeStruct((PPAD, D), jnp.float32),
        ],
        mesh=mesh,
        scratch_types=[
            pltpu.VMEM((PG_C,), jnp.int32),
            pltpu.VMEM((PG_C,), jnp.int32),
            pltpu.VMEM((PG_C, D), jnp.float32),
            pltpu.VMEM((PG_C, D), jnp.float32),
            pltpu.SemaphoreType.DMA,
        ],
        compiler_params=_SC_PARAMS,
    )


def _dot_t(a, b):
    return lax.dot_general(a, b, (((1,), (1,)), ((), ())),
                           preferred_element_type=jnp.float32)


def _tc_layer1_body(parts, dcol, x, w1l, b1l, w1r, gamma, beta, h):
    s = parts[0, :N] + parts[1, :N]
    agg = s / jnp.maximum(dcol[...], 1.0)
    t = _dot_t(agg, w1l[...]) + b1l[...] + _dot_t(x[...], w1r[...])
    mu = jnp.mean(t, axis=0, keepdims=True)
    var = jnp.mean((t - mu) ** 2, axis=0, keepdims=True)
    hh = gamma[...] * (t - mu) * lax.rsqrt(var + EPS) + beta[...]
    h[...] = jnp.maximum(hh, 0.0)


def _tc_layer2_body(parts, dcol, h, w2l, b2l, w2r, w3a, w3b, b3, u, v):
    s = parts[0, :N] + parts[1, :N]
    agg = s / jnp.maximum(dcol[...], 1.0)
    z = _dot_t(agg, w2l[...]) + b2l[...] + _dot_t(h[...], w2r[...])
    u[...] = _dot_t(z, w3a[...]) + b3[...]
    v[...] = _dot_t(z, w3b[...])


HBLK = 2048


def _tc_head_body(gu, gv, w4p, b4p, w5a, o):
    h1 = jnp.maximum(gu[...] + gv[...], 0.0)
    h2 = jnp.maximum(_dot_t(h1, w4p[...]) + b4p[...], 0.0)
    o[...] = jax.nn.sigmoid(_dot_t(h2, w5a[...]))


def kernel(x, edge_index, edge_pairs, W1l, b1l, W1r, gamma, beta,
           W2l, b2l, W2r, W3, b3, W4, b4, W5, b5):
    npd = EPAD - E
    src = jnp.concatenate([edge_index[0], jnp.arange(npd, dtype=jnp.int32) % N])
    dst = jnp.concatenate(
        [edge_index[1], N + jnp.arange(npd, dtype=jnp.int32) % (NPAD - N)])
    zfeat = jnp.zeros((SEG_C, D), jnp.float32)

    parts1, degflat = _get_segsum(True)(x, src, dst, zfeat)
    dcol = degflat.reshape(NW, NPAD).sum(axis=0)[:N, None]

    h = pl.pallas_call(
        _tc_layer1_body,
        out_shape=jax.ShapeDtypeStruct((N, D), jnp.float32),
    )(parts1, dcol, x, W1l, b1l.reshape(1, D), W1r,
      gamma.reshape(1, D), beta.reshape(1, D))

    (parts2,) = _get_segsum(False)(h, src, dst, zfeat)

    u, v = pl.pallas_call(
        _tc_layer2_body,
        out_shape=[
            jax.ShapeDtypeStruct((N, D), jnp.float32),
            jax.ShapeDtypeStruct((N, D), jnp.float32),
        ],
    )(parts2, dcol, h, W2l, b2l.reshape(1, D), W2r,
      W3[:, :D], W3[:, D:], b3.reshape(1, D))

    pad = jnp.zeros((PPAD - P,), jnp.int32)
    p0 = jnp.concatenate([edge_pairs[0], pad])
    p1 = jnp.concatenate([edge_pairs[1], pad])
    gu, gv = _get_pair_gather()(u, v, p0, p1)

    w4p = jnp.concatenate([W4, jnp.zeros((D // 2, D), jnp.float32)], axis=0)
    b4p = jnp.concatenate(
        [b4, jnp.ones((1,), jnp.float32),
         jnp.zeros((D // 2 - 1,), jnp.float32)]).reshape(1, D)
    w5a = jnp.concatenate(
        [W5, b5.reshape(1, 1), jnp.zeros((1, D // 2 - 1), jnp.float32)],
        axis=1)

    o = pl.pallas_call(
        _tc_head_body,
        grid=(PPAD // HBLK,),
        in_specs=[
            pl.BlockSpec((HBLK, D), lambda i: (i, 0)),
            pl.BlockSpec((HBLK, D), lambda i: (i, 0)),
            pl.BlockSpec((D, D), lambda i: (0, 0)),
            pl.BlockSpec((1, D), lambda i: (0, 0)),
            pl.BlockSpec((1, D), lambda i: (0, 0)),
        ],
        out_specs=pl.BlockSpec((HBLK, 1), lambda i: (i, 0)),
        out_shape=jax.ShapeDtypeStruct((PPAD, 1), jnp.float32),
    )(gu, gv, w4p, b4p, w5a)

    return o[:P]


# speedup vs baseline: 3.7573x; 2.0495x over previous
import functools

import jax
import jax.numpy as jnp
from jax import lax
from jax.experimental import pallas as pl
from jax.experimental.pallas import tpu as pltpu
from jax.experimental.pallas import tpu_sc as plsc

N = 10000
D = 128
E = 640000
P = 100000
EPS = 1e-5

NC = 2
NS = 16
NW = NC * NS

SEG_C = 128
SEG_CH0 = 160
SEG_CH1 = 160
EPAD = NS * (SEG_CH0 + SEG_CH1) * SEG_C
NPAD = 10240
RPT = NPAD // NS
ZITER = RPT // SEG_C

PG_C = 80
PG_CH0 = 40
PG_CH1 = 40
PPAD = NS * (PG_CH0 + PG_CH1) * PG_C

_SC_PARAMS = pltpu.CompilerParams(use_tc_tiling_on_sc=False,
                                  needs_layout_passes=False)


def _sc_segsum_body(compute_deg, *refs):
    if compute_deg:
        (feats, src, dst, zfeat, parts, degflat,
         acc, is0, is1, is2, is3, id0, id1, id2, id3, rows0, rows1, dpriv,
         semis, semid, semg, semc) = refs
    else:
        (feats, src, dst, zfeat, parts,
         acc, is0, is1, is2, is3, id0, id1, id2, id3, rows0, rows1,
         semis, semid, semg, semc) = refs
    ISS = [is0, is1, is2, is3]
    IDS = [id0, id1, id2, id3]
    ROWS = [rows0, rows1]
    cid = lax.axis_index("c")
    sid = lax.axis_index("s")
    wid = cid * NS + sid
    cc = jnp.where(cid == 0, SEG_CH0, SEG_CH1)
    base = SEG_C * (cid * NS * SEG_CH0 + sid * cc)
    g4 = cc // 4

    pltpu.sync_copy(zfeat, rows0)
    for j in range(ZITER):
        pltpu.sync_copy(rows0, acc.at[pl.ds(sid * RPT + j * SEG_C, SEG_C)])
    if compute_deg:
        def zero_deg(i, carry):
            dpriv[pl.ds(i * 16, 16)] = jnp.zeros((16,), jnp.float32)
            return carry
        lax.fori_loop(0, NPAD // 16, zero_deg, 0)
    plsc.subcore_barrier()

    ones16 = jnp.ones((16,), jnp.float32)

    def off_of(g):
        return pl.multiple_of(base + g * SEG_C, 8)

    pltpu.sync_copy(src.at[pl.ds(off_of(0), SEG_C)], ISS[0])
    pltpu.sync_copy(dst.at[pl.ds(off_of(0), SEG_C)], IDS[0])
    pltpu.async_copy(src.at[pl.ds(off_of(1), SEG_C)], ISS[1], semis.at[1])
    pltpu.async_copy(dst.at[pl.ds(off_of(1), SEG_C)], IDS[1], semid.at[1])
    pltpu.async_copy(feats.at[ISS[0]], ROWS[0], semg.at[0])

    def quad(t, carry):
        for u in range(4):
            b2 = u % 2
            g = 4 * t + u
            pltpu.make_async_copy(feats.at[ISS[u]], ROWS[b2],
                                  semg.at[b2]).wait()
            pltpu.async_copy(ROWS[b2], acc.at[IDS[u]], semc, add=True)
            if compute_deg:
                for k in range(SEG_C // 16):
                    dv = IDS[u][pl.ds(k * 16, 16)]
                    plsc.addupdate_scatter(dpriv, [dv], ones16)
            def wait_sc():
                pltpu.make_async_copy(ROWS[(u - 1) % 2],
                                      acc.at[IDS[(u - 1) % 4]], semc).wait()
            if u == 0:
                pl.when(t > 0)(wait_sc)
            else:
                wait_sc()
            def pref():
                off2 = off_of(g + 2)
                pltpu.async_copy(src.at[pl.ds(off2, SEG_C)],
                                 ISS[(u + 2) % 4], semis.at[(u + 2) % 4])
                pltpu.async_copy(dst.at[pl.ds(off2, SEG_C)],
                                 IDS[(u + 2) % 4], semid.at[(u + 2) % 4])
            if u < 2:
                pref()
            else:
                pl.when(t < g4 - 1)(pref)
            def nxt():
                off1 = off_of(g + 1)
                pltpu.make_async_copy(src.at[pl.ds(off1, SEG_C)],
                                      ISS[(u + 1) % 4],
                                      semis.at[(u + 1) % 4]).wait()
                pltpu.make_async_copy(dst.at[pl.ds(off1, SEG_C)],
                                      IDS[(u + 1) % 4],
                                      semid.at[(u + 1) % 4]).wait()
                pltpu.async_copy(feats.at[ISS[(u + 1) % 4]], ROWS[(u + 1) % 2],
                                 semg.at[(u + 1) % 2])
            if u == 3:
                pl.when(t < g4 - 1)(nxt)
            else:
                nxt()
        return carry

    lax.fori_loop(0, g4, quad, 0)
    pltpu.make_async_copy(ROWS[1], acc.at[IDS[3]], semc).wait()
    plsc.subcore_barrier()

    for j in range(ZITER):
        sl = pl.ds(sid * RPT + j * SEG_C, SEG_C)
        pltpu.sync_copy(acc.at[sl], rows0)
        pltpu.sync_copy(rows0, parts.at[cid].at[sl])
    if compute_deg:
        pltpu.sync_copy(dpriv, degflat.at[pl.ds(wid * NPAD, NPAD)])


@functools.lru_cache(maxsize=None)
def _get_segsum(compute_deg):
    mesh = plsc.VectorSubcoreMesh(core_axis_name="c", subcore_axis_name="s")
    out_type = [jax.ShapeDtypeStruct((NC, NPAD, D), jnp.float32)]
    if compute_deg:
        out_type.append(jax.ShapeDtypeStruct((NW * NPAD,), jnp.float32))
    scratch = [pltpu.VMEM_SHARED((NPAD, D), jnp.float32)]
    scratch += [pltpu.VMEM((SEG_C,), jnp.int32)] * 8
    scratch += [pltpu.VMEM((SEG_C, D), jnp.float32)] * 2
    if compute_deg:
        scratch.append(pltpu.VMEM((NPAD,), jnp.float32))
    scratch += [
        pltpu.SemaphoreType.DMA((4,)),
        pltpu.SemaphoreType.DMA((4,)),
        pltpu.SemaphoreType.DMA((2,)),
        pltpu.SemaphoreType.DMA,
    ]
    return pl.kernel(
        functools.partial(_sc_segsum_body, compute_deg),
        out_type=out_type,
        mesh=mesh,
        scratch_types=scratch,
        compiler_params=_SC_PARAMS,
    )


def _sc_pairs_body(u, v, p0, p1, gu, gv,
                   i00, i01, i02, i03, i10, i11, i12, i13,
                   bu0, bu1, bv0, bv1,
                   sem0, sem1, semgu, semgv, semw):
    I0 = [i00, i01, i02, i03]
    I1 = [i10, i11, i12, i13]
    BU = [bu0, bu1]
    BV = [bv0, bv1]
    cid = lax.axis_index("c")
    sid = lax.axis_index("s")
    cc = jnp.where(cid == 0, PG_CH0, PG_CH1)
    base = PG_C * (cid * NS * PG_CH0 + sid * cc)
    g4 = cc // 4

    def off_of(g):
        return pl.multiple_of(base + g * PG_C, 8)

    pltpu.sync_copy(p0.at[pl.ds(off_of(0), PG_C)], I0[0])
    pltpu.sync_copy(p1.at[pl.ds(off_of(0), PG_C)], I1[0])
    pltpu.async_copy(p0.at[pl.ds(off_of(1), PG_C)], I0[1], sem0.at[1])
    pltpu.async_copy(p1.at[pl.ds(off_of(1), PG_C)], I1[1], sem1.at[1])
    pltpu.async_copy(u.at[I0[0]], BU[0], semgu.at[0])
    pltpu.async_copy(v.at[I1[0]], BV[0], semgv.at[0])

    def quad(t, carry):
        for uu in range(4):
            b2 = uu % 2
            g = 4 * t + uu
            sl = pl.ds(off_of(g), PG_C)
            pltpu.make_async_copy(u.at[I0[uu]], BU[b2], semgu.at[b2]).wait()
            pltpu.make_async_copy(v.at[I1[uu]], BV[b2], semgv.at[b2]).wait()
            pltpu.async_copy(BU[b2], gu.at[sl], semw)
            pltpu.async_copy(BV[b2], gv.at[sl], semw)
            def wait_w():
                slp = pl.ds(off_of(g - 1), PG_C)
                pltpu.make_async_copy(BU[(uu - 1) % 2], gu.at[slp],
                                      semw).wait()
                pltpu.make_async_copy(BV[(uu - 1) % 2], gv.at[slp],
                                      semw).wait()
            if uu == 0:
                pl.when(t > 0)(wait_w)
            else:
                wait_w()
            def pref():
                off2 = off_of(g + 2)
                pltpu.async_copy(p0.at[pl.ds(off2, PG_C)], I0[(uu + 2) % 4],
                                 sem0.at[(uu + 2) % 4])
                pltpu.async_copy(p1.at[pl.ds(off2, PG_C)], I1[(uu + 2) % 4],
                                 sem1.at[(uu + 2) % 4])
            if uu < 2:
                pref()
            else:
                pl.when(t < g4 - 1)(pref)
            def nxt():
                off1 = off_of(g + 1)
                pltpu.make_async_copy(p0.at[pl.ds(off1, PG_C)],
                                      I0[(uu + 1) % 4],
                                      sem0.at[(uu + 1) % 4]).wait()
                pltpu.make_async_copy(p1.at[pl.ds(off1, PG_C)],
                                      I1[(uu + 1) % 4],
                                      sem1.at[(uu + 1) % 4]).wait()
                pltpu.async_copy(u.at[I0[(uu + 1) % 4]], BU[(uu + 1) % 2],
                                 semgu.at[(uu + 1) % 2])
                pltpu.async_copy(v.at[I1[(uu + 1) % 4]], BV[(uu + 1) % 2],
                                 semgv.at[(uu + 1) % 2])
            if uu == 3:
                pl.when(t < g4 - 1)(nxt)
            else:
                nxt()
        return carry

    lax.fori_loop(0, g4, quad, 0)
    slp = pl.ds(off_of(cc - 1), PG_C)
    pltpu.make_async_copy(BU[1], gu.at[slp], semw).wait()
    pltpu.make_async_copy(BV[1], gv.at[slp], semw).wait()


@functools.lru_cache(maxsize=None)
def _get_pair_gather():
    mesh = plsc.VectorSubcoreMesh(core_axis_name="c", subcore_axis_name="s")
    return pl.kernel(
        _sc_pairs_body,
        out_type=[
            jax.ShapeDtypeStruct((PPAD, D), jnp.float32),
            jax.ShapeDtypeStruct((PPAD, D), jnp.float32),
        ],
        mesh=mesh,
        scratch_types=(
            [pltpu.VMEM((PG_C,), jnp.int32)] * 8
            + [pltpu.VMEM((PG_C, D), jnp.float32)] * 4
            + [pltpu.SemaphoreType.DMA((4,)),
               pltpu.SemaphoreType.DMA((4,)),
               pltpu.SemaphoreType.DMA((2,)),
               pltpu.SemaphoreType.DMA((2,)),
               pltpu.SemaphoreType.DMA]
        ),
        compiler_params=_SC_PARAMS,
    )


def _dot_t(a, b):
    return lax.dot_general(a, b, (((1,), (1,)), ((), ())),
                           preferred_element_type=jnp.float32)


def _tc_layer1_body(parts, dcol, x, w1l, b1l, w1r, gamma, beta, h):
    s = parts[0, :N] + parts[1, :N]
    agg = s / jnp.maximum(dcol[...], 1.0)
    t = _dot_t(agg, w1l[...]) + b1l[...] + _dot_t(x[...], w1r[...])
    mu = jnp.mean(t, axis=0, keepdims=True)
    var = jnp.mean((t - mu) ** 2, axis=0, keepdims=True)
    hh = gamma[...] * (t - mu) * lax.rsqrt(var + EPS) + beta[...]
    h[...] = jnp.maximum(hh, 0.0)


def _tc_layer2_body(parts, dcol, h, w2l, b2l, w2r, w3a, w3b, b3, u, v):
    s = parts[0, :N] + parts[1, :N]
    agg = s / jnp.maximum(dcol[...], 1.0)
    z = _dot_t(agg, w2l[...]) + b2l[...] + _dot_t(h[...], w2r[...])
    u[...] = _dot_t(z, w3a[...]) + b3[...]
    v[...] = _dot_t(z, w3b[...])


HBLK = 2048


def _tc_head_body(gu, gv, w4p, b4p, w5a, o):
    h1 = jnp.maximum(gu[...] + gv[...], 0.0)
    h2 = jnp.maximum(_dot_t(h1, w4p[...]) + b4p[...], 0.0)
    o[...] = jax.nn.sigmoid(_dot_t(h2, w5a[...]))


def kernel(x, edge_index, edge_pairs, W1l, b1l, W1r, gamma, beta,
           W2l, b2l, W2r, W3, b3, W4, b4, W5, b5):
    npd = EPAD - E
    src = jnp.concatenate([edge_index[0], jnp.arange(npd, dtype=jnp.int32) % N])
    dst = jnp.concatenate(
        [edge_index[1], N + jnp.arange(npd, dtype=jnp.int32) % (NPAD - N)])
    zfeat = jnp.zeros((SEG_C, D), jnp.float32)

    parts1, degflat = _get_segsum(True)(x, src, dst, zfeat)
    dcol = degflat.reshape(NW, NPAD).sum(axis=0)[:N, None]

    h = pl.pallas_call(
        _tc_layer1_body,
        out_shape=jax.ShapeDtypeStruct((N, D), jnp.float32),
    )(parts1, dcol, x, W1l, b1l.reshape(1, D), W1r,
      gamma.reshape(1, D), beta.reshape(1, D))

    (parts2,) = _get_segsum(False)(h, src, dst, zfeat)

    u, v = pl.pallas_call(
        _tc_layer2_body,
        out_shape=[
            jax.ShapeDtypeStruct((N, D), jnp.float32),
            jax.ShapeDtypeStruct((N, D), jnp.float32),
        ],
    )(parts2, dcol, h, W2l, b2l.reshape(1, D), W2r,
      W3[:, :D], W3[:, D:], b3.reshape(1, D))

    pad = jnp.arange(PPAD - P, dtype=jnp.int32) % N
    p0 = jnp.concatenate([edge_pairs[0], pad])
    p1 = jnp.concatenate([edge_pairs[1], pad])
    gu, gv = _get_pair_gather()(u, v, p0, p1)

    w4p = jnp.concatenate([W4, jnp.zeros((D // 2, D), jnp.float32)], axis=0)
    b4p = jnp.concatenate(
        [b4, jnp.ones((1,), jnp.float32),
         jnp.zeros((D // 2 - 1,), jnp.float32)]).reshape(1, D)
    w5a = jnp.concatenate(
        [W5, b5.reshape(1, 1), jnp.zeros((1, D // 2 - 1), jnp.float32)],
        axis=1)

    o = pl.pallas_call(
        _tc_head_body,
        grid=(PPAD // HBLK,),
        in_specs=[
            pl.BlockSpec((HBLK, D), lambda i: (i, 0)),
            pl.BlockSpec((HBLK, D), lambda i: (i, 0)),
            pl.BlockSpec((D, D), lambda i: (0, 0)),
            pl.BlockSpec((1, D), lambda i: (0, 0)),
            pl.BlockSpec((1, D), lambda i: (0, 0)),
        ],
        out_specs=pl.BlockSpec((HBLK, 1), lambda i: (i, 0)),
        out_shape=jax.ShapeDtypeStruct((PPAD, 1), jnp.float32),
    )(gu, gv, w4p, b4p, w5a)

    return o[:P]
